# Initial kernel scaffold; baseline (speedup 1.0000x reference)
import jax
import jax.numpy as jnp
from jax import lax
from jax.experimental import pallas as pl
from jax.experimental.pallas import tpu as pltpu
from jax.experimental.pallas import tpu_sc as plsc

DIM = 48
CHN = 8
NROWS = DIM * DIM * DIM * DIM
C = 128

_info = plsc.get_sparse_core_info()
NUM_CORES = _info.num_cores
NUM_SUBCORES = _info.num_subcores
LANES = _info.num_lanes
NW = NUM_CORES * NUM_SUBCORES

_CORNERS = [(o3, o2, o1, o0)
            for o3 in (0, 1) for o2 in (0, 1) for o1 in (0, 1) for o0 in (0, 1)]
_OFFS = [((o3 * DIM + o2) * DIM + o1) * DIM + o0 for (o3, o2, o1, o0) in _CORNERS]


def _body(nchunks, uvwq_hbm, table_hbm, out_hbm, uv_v, idx_v, w_v, rows_v,
          out_v, sem):
    wid = lax.axis_index("s") * NUM_CORES + lax.axis_index("c")
    base = wid * (nchunks * C)
    iota = lax.iota(jnp.int32, LANES)

    def chunk(k, carry):
        qbase = base + k * C
        for d in range(4):
            pltpu.sync_copy(uvwq_hbm.at[d, pl.ds(qbase, C)], uv_v.at[d])

        def phase1(j, carry1):
            sl = pl.ds(j * LANES, LANES)
            f = [uv_v[d, sl] * float(DIM - 1) for d in range(4)]
            ii = [fd.astype(jnp.int32) for fd in f]
            fr = [f[d] - ii[d].astype(jnp.float32) for d in range(4)]
            r0 = ((ii[3] * DIM + ii[2]) * DIM + ii[1]) * DIM + ii[0]
            t = [(1.0 - fr[d], fr[d]) for d in range(4)]
            for c, (o3, o2, o1, o0) in enumerate(_CORNERS):
                idx_v[c, sl] = r0 + _OFFS[c]
                w_v[c, sl] = (t[3][o3] * t[2][o2]) * (t[1][o1] * t[0][o0])
            return carry1

        lax.fori_loop(0, C // LANES, phase1, 0, unroll=False)

        copies = [
            pltpu.async_copy(table_hbm.at[idx_v.at[c]], rows_v.at[c], sem)
            for c in range(16)
        ]
        for cp in copies:
            cp.wait()

        def phase2(j, carry2):
            sl = pl.ds(j * LANES, LANES)
            qidx = j * LANES + iota
            wv = [w_v[c, sl] for c in range(16)]
            for ch in range(CHN):
                chv = jnp.full((LANES,), ch, jnp.int32)
                acc = None
                for c in range(16):
                    cc = jnp.full((LANES,), c, jnp.int32)
                    x = plsc.load_gather(rows_v, [cc, qidx, chv])
                    acc = wv[c] * x if acc is None else acc + wv[c] * x
                plsc.store_scatter(out_v, [qidx, chv], acc)
            return carry2

        lax.fori_loop(0, C // LANES, phase2, 0, unroll=False)

        pltpu.sync_copy(out_v, out_hbm.at[pl.ds(qbase, C)])
        return carry

    lax.fori_loop(0, nchunks, chunk, 0, unroll=False)


def kernel(uvwqList, array4D):
    n = uvwqList.shape[0]
    nchunks = -(-n // (NW * C))
    npad = NW * nchunks * C

    table = array4D.reshape(NROWS, CHN)
    uvwq_t = jnp.transpose(uvwqList)
    uvwq_t = jnp.pad(uvwq_t, ((0, 0), (0, npad - n)))

    mesh = plsc.VectorSubcoreMesh(core_axis_name="c", subcore_axis_name="s")
    body = lambda *refs: _body(nchunks, *refs)
    out = pl.kernel(
        body,
        mesh=mesh,
        out_type=jax.ShapeDtypeStruct((npad, CHN), jnp.float32),
        scratch_types=[
            pltpu.VMEM((4, C), jnp.float32),
            pltpu.VMEM((16, C), jnp.int32),
            pltpu.VMEM((16, C), jnp.float32),
            pltpu.VMEM((16, C, CHN), jnp.float32),
            pltpu.VMEM((C, CHN), jnp.float32),
            pltpu.SemaphoreType.DMA,
        ],
    )(uvwq_t, table)
    return out[:n]


# speedup vs baseline: 1.1098x; 1.1098x over previous
import jax
import jax.numpy as jnp
from jax import lax
from jax.experimental import pallas as pl
from jax.experimental.pallas import tpu as pltpu
from jax.experimental.pallas import tpu_sc as plsc

DIM = 48
CHN = 8
NROWS = DIM * DIM * DIM * DIM
C = 128

NUM_CORES = 2
NUM_SUBCORES = 16
LANES = 16
NW = NUM_CORES * NUM_SUBCORES

_CORNERS = [(o3, o2, o1, o0)
            for o3 in (0, 1) for o2 in (0, 1) for o1 in (0, 1) for o0 in (0, 1)]
_OFFS = [((o3 * DIM + o2) * DIM + o1) * DIM + o0 for (o3, o2, o1, o0) in _CORNERS]


def _body(nchunks, uvwq_hbm, table_hbm, out_hbm, uv_v, idx_v, w_v, rows_v,
          out_v, sem):
    wid = lax.axis_index("s") * NUM_CORES + lax.axis_index("c")
    base = wid * (nchunks * C)
    iota = lax.iota(jnp.int32, LANES)

    def chunk(k, carry):
        qbase = base + k * C
        for d in range(4):
            pltpu.sync_copy(uvwq_hbm.at[d, pl.ds(qbase, C)], uv_v.at[d])

        def phase1(j, carry1):
            sl = pl.ds(j * LANES, LANES)
            f = [uv_v[d, sl] * float(DIM - 1) for d in range(4)]
            ii = [fd.astype(jnp.int32) for fd in f]
            fr = [f[d] - ii[d].astype(jnp.float32) for d in range(4)]
            r0 = ((ii[3] * DIM + ii[2]) * DIM + ii[1]) * DIM + ii[0]
            t = [(1.0 - fr[d], fr[d]) for d in range(4)]
            for c, (o3, o2, o1, o0) in enumerate(_CORNERS):
                idx_v[c, sl] = r0 + _OFFS[c]
                w_v[c, sl] = (t[3][o3] * t[2][o2]) * (t[1][o1] * t[0][o0])
            return carry1

        lax.fori_loop(0, C // LANES, phase1, 0, unroll=False)

        copies = [
            pltpu.async_copy(table_hbm.at[idx_v.at[c]], rows_v.at[c], sem)
            for c in range(16)
        ]
        for cp in copies:
            cp.wait()

        def phase2(j, carry2):
            sl = pl.ds(j * LANES, LANES)
            qidx = j * LANES + iota
            wv = [w_v[c, sl] for c in range(16)]
            for ch in range(CHN):
                chv = jnp.full((LANES,), ch, jnp.int32)
                acc = None
                for c in range(16):
                    cc = jnp.full((LANES,), c, jnp.int32)
                    x = plsc.load_gather(rows_v, [cc, qidx, chv])
                    acc = wv[c] * x if acc is None else acc + wv[c] * x
                plsc.store_scatter(out_v, [qidx, chv], acc)
            return carry2

        lax.fori_loop(0, C // LANES, phase2, 0, unroll=False)

        pltpu.sync_copy(out_v, out_hbm.at[pl.ds(qbase, C)])
        return carry

    lax.fori_loop(0, nchunks, chunk, 0, unroll=False)


def kernel(uvwqList, array4D):
    n = uvwqList.shape[0]
    nchunks = -(-n // (NW * C))
    npad = NW * nchunks * C

    table = array4D.reshape(NROWS, CHN)
    uvwq_t = jnp.transpose(uvwqList)
    uvwq_t = jnp.pad(uvwq_t, ((0, 0), (0, npad - n)))

    mesh = plsc.VectorSubcoreMesh(core_axis_name="c", subcore_axis_name="s")
    body = lambda *refs: _body(nchunks, *refs)
    out = pl.kernel(
        body,
        mesh=mesh,
        compiler_params=pltpu.CompilerParams(
            use_tc_tiling_on_sc=False, needs_layout_passes=False),
        out_type=jax.ShapeDtypeStruct((npad, CHN), jnp.float32),
        scratch_types=[
            pltpu.VMEM((4, C), jnp.float32),
            pltpu.VMEM((16, C), jnp.int32),
            pltpu.VMEM((16, C), jnp.float32),
            pltpu.VMEM((16, C, CHN), jnp.float32),
            pltpu.VMEM((C, CHN), jnp.float32),
            pltpu.SemaphoreType.DMA,
        ],
    )(uvwq_t, table)
    return out[:n]


# speedup vs baseline: 1.1394x; 1.0266x over previous
import jax
import jax.numpy as jnp
from jax import lax
from jax.experimental import pallas as pl
from jax.experimental.pallas import tpu as pltpu
from jax.experimental.pallas import tpu_sc as plsc

DIM = 48
CHN = 8
NROWS = DIM * DIM * DIM * DIM
C = 128

NUM_CORES = 2
NUM_SUBCORES = 16
LANES = 16
NW = NUM_CORES * NUM_SUBCORES

_CORNERS = [(o3, o2, o1, o0)
            for o3 in (0, 1) for o2 in (0, 1) for o1 in (0, 1) for o0 in (0, 1)]
_OFFS = [((o3 * DIM + o2) * DIM + o1) * DIM + o0 for (o3, o2, o1, o0) in _CORNERS]


def _make_body(nchunks, n):
    def body(uvwq_hbm, table_hbm, out_hbm, uv_v, idx_v, w_v, rows_v, out_v,
             sem_uv, sem_g, sem_out):
        wid = lax.axis_index("s") * NUM_CORES + lax.axis_index("c")
        nj = (nchunks - wid + NW - 1) // NW
        iota = lax.iota(jnp.int32, LANES)

        def qbase_of(j):
            t = wid + j * NW
            return jnp.minimum(t * C, n - C)

        def fire_uv(j, p):
            pltpu.async_copy(uvwq_hbm.at[pl.ds(qbase_of(j), C), :],
                             uv_v.at[p], sem_uv.at[p])

        def wait_uv(p):
            pltpu.make_async_copy(uvwq_hbm.at[pl.ds(0, C), :],
                                  uv_v.at[p], sem_uv.at[p]).wait()

        def phase1_and_fire(p):
            def phase1(j2, carry1):
                sl = pl.ds(j2 * LANES, LANES)
                qidx = j2 * LANES + iota
                f = [
                    plsc.load_gather(
                        uv_v.at[p], [qidx, jnp.full((LANES,), d, jnp.int32)])
                    * float(DIM - 1)
                    for d in range(4)
                ]
                ii = [fd.astype(jnp.int32) for fd in f]
                fr = [f[d] - ii[d].astype(jnp.float32) for d in range(4)]
                r0 = ((ii[3] * DIM + ii[2]) * DIM + ii[1]) * DIM + ii[0]
                t = [(1.0 - fr[d], fr[d]) for d in range(4)]
                for c, (o3, o2, o1, o0) in enumerate(_CORNERS):
                    idx_v[p, c, sl] = r0 + _OFFS[c]
                    w_v[p, c, sl] = (t[3][o3] * t[2][o2]) * (t[1][o1] * t[0][o0])
                return carry1

            lax.fori_loop(0, C // LANES, phase1, 0, unroll=False)
            for c in range(16):
                pltpu.async_copy(table_hbm.at[idx_v.at[p, c]],
                                 rows_v.at[p, c], sem_g.at[p])

        def wait_gathers(p):
            for c in range(16):
                pltpu.make_async_copy(table_hbm.at[idx_v.at[p, c]],
                                      rows_v.at[p, c], sem_g.at[p]).wait()

        def phase2(p):
            def inner(j2, carry2):
                sl = pl.ds(j2 * LANES, LANES)
                qidx = j2 * LANES + iota
                wv = [w_v[p, c, sl] for c in range(16)]
                for ch in range(CHN):
                    chv = jnp.full((LANES,), ch, jnp.int32)
                    acc = None
                    for c in range(16):
                        cc = jnp.full((LANES,), c, jnp.int32)
                        x = plsc.load_gather(rows_v.at[p], [cc, qidx, chv])
                        acc = wv[c] * x if acc is None else acc + wv[c] * x
                    plsc.store_scatter(out_v.at[p], [qidx, chv], acc)
                return carry2

            lax.fori_loop(0, C // LANES, inner, 0, unroll=False)

        def fire_out(j, p):
            pltpu.async_copy(out_v.at[p], out_hbm.at[pl.ds(qbase_of(j), C)],
                             sem_out.at[p])

        def wait_out(p):
            pltpu.make_async_copy(out_v.at[p], out_hbm.at[pl.ds(0, C)],
                                  sem_out.at[p]).wait()

        fire_uv(0, 0)

        @pl.when(nj > 1)
        def _():
            fire_uv(1, 1)

        wait_uv(0)
        phase1_and_fire(0)

        def step(j, carry):
            p = lax.rem(j, 2)
            pn = lax.rem(j + 1, 2)

            @pl.when(j + 2 < nj)
            def _():
                fire_uv(j + 2, p)

            @pl.when(j + 1 < nj)
            def _():
                wait_uv(pn)
                phase1_and_fire(pn)

            wait_gathers(p)

            @pl.when(j >= 2)
            def _():
                wait_out(p)

            phase2(p)
            fire_out(j, p)
            return carry

        lax.fori_loop(0, nj, step, 0, unroll=False)

        @pl.when(nj >= 2)
        def _():
            wait_out(lax.rem(nj, 2))

        wait_out(lax.rem(nj + 1, 2))

    return body


def kernel(uvwqList, array4D):
    n = uvwqList.shape[0]
    nchunks = -(-n // C)
    assert n >= C and n % 8 == 0

    table = array4D.reshape(NROWS, CHN)

    mesh = plsc.VectorSubcoreMesh(core_axis_name="c", subcore_axis_name="s")
    out = pl.kernel(
        _make_body(nchunks, n),
        mesh=mesh,
        compiler_params=pltpu.CompilerParams(
            use_tc_tiling_on_sc=False, needs_layout_passes=False),
        out_type=jax.ShapeDtypeStruct((n, CHN), jnp.float32),
        scratch_types=[
            pltpu.VMEM((2, C, 4), jnp.float32),
            pltpu.VMEM((2, 16, C), jnp.int32),
            pltpu.VMEM((2, 16, C), jnp.float32),
            pltpu.VMEM((2, 16, C, CHN), jnp.float32),
            pltpu.VMEM((2, C, CHN), jnp.float32),
            pltpu.SemaphoreType.DMA((2,)),
            pltpu.SemaphoreType.DMA((2,)),
            pltpu.SemaphoreType.DMA((2,)),
        ],
    )(uvwqList, table)
    return out


# speedup vs baseline: 1.1647x; 1.0223x over previous
import jax
import jax.numpy as jnp
from jax import lax
from jax.experimental import pallas as pl
from jax.experimental.pallas import tpu as pltpu
from jax.experimental.pallas import tpu_sc as plsc

DIM = 48
CHN = 8
NROWS = DIM * DIM * DIM * DIM
C = 128

NUM_CORES = 2
NUM_SUBCORES = 16
LANES = 16
NW = NUM_CORES * NUM_SUBCORES

_CORNERS = [(o3, o2, o1, o0)
            for o3 in (0, 1) for o2 in (0, 1) for o1 in (0, 1) for o0 in (0, 1)]
_OFFS = [((o3 * DIM + o2) * DIM + o1) * DIM + o0 for (o3, o2, o1, o0) in _CORNERS]


def _make_body(nchunks, n):
    def body(uvwq_hbm, table_hbm, out_hbm, uv_v, idx_v, w_v, rows_v, out_v,
             sem_uv, sem_g, sem_out):
        wid = lax.axis_index("s") * NUM_CORES + lax.axis_index("c")
        nj = (nchunks - wid + NW - 1) // NW
        iota = lax.iota(jnp.int32, LANES)

        def qbase_of(j):
            t = wid + j * NW
            return jnp.minimum(t * C, n - C)

        def fire_uv(j, p):
            pltpu.async_copy(uvwq_hbm.at[pl.ds(qbase_of(j) * 4, C * 4)],
                             uv_v.at[p], sem_uv.at[p])

        def wait_uv(p):
            pltpu.make_async_copy(uvwq_hbm.at[pl.ds(0, C * 4)],
                                  uv_v.at[p], sem_uv.at[p]).wait()

        def phase1_and_fire(p):
            def phase1(j2, carry1):
                sl = pl.ds(j2 * LANES, LANES)
                qidx = j2 * LANES + iota
                f = [
                    plsc.load_gather(
                        uv_v.at[p],
                        [qidx * 4 + jnp.full((LANES,), d, jnp.int32)])
                    * float(DIM - 1)
                    for d in range(4)
                ]
                ii = [fd.astype(jnp.int32) for fd in f]
                fr = [f[d] - ii[d].astype(jnp.float32) for d in range(4)]
                r0 = ((ii[3] * DIM + ii[2]) * DIM + ii[1]) * DIM + ii[0]
                t = [(1.0 - fr[d], fr[d]) for d in range(4)]
                for c, (o3, o2, o1, o0) in enumerate(_CORNERS):
                    idx_v[p, c, sl] = r0 + _OFFS[c]
                    w_v[p, c, sl] = (t[3][o3] * t[2][o2]) * (t[1][o1] * t[0][o0])
                return carry1

            lax.fori_loop(0, C // LANES, phase1, 0, unroll=False)
            for c in range(16):
                pltpu.async_copy(table_hbm.at[idx_v.at[p, c]],
                                 rows_v.at[p, c], sem_g.at[p])

        def wait_gathers(p):
            for c in range(16):
                pltpu.make_async_copy(table_hbm.at[idx_v.at[p, c]],
                                      rows_v.at[p, c], sem_g.at[p]).wait()

        def phase2(p):
            def inner(j2, carry2):
                sl = pl.ds(j2 * LANES, LANES)
                qidx = j2 * LANES + iota
                wv = [w_v[p, c, sl] for c in range(16)]
                for ch in range(CHN):
                    chv = jnp.full((LANES,), ch, jnp.int32)
                    acc = None
                    for c in range(16):
                        cc = jnp.full((LANES,), c, jnp.int32)
                        x = plsc.load_gather(rows_v.at[p], [cc, qidx, chv])
                        acc = wv[c] * x if acc is None else acc + wv[c] * x
                    plsc.store_scatter(out_v.at[p], [qidx, chv], acc)
                return carry2

            lax.fori_loop(0, C // LANES, inner, 0, unroll=False)

        def fire_out(j, p):
            pltpu.async_copy(out_v.at[p], out_hbm.at[pl.ds(qbase_of(j), C)],
                             sem_out.at[p])

        def wait_out(p):
            pltpu.make_async_copy(out_v.at[p], out_hbm.at[pl.ds(0, C)],
                                  sem_out.at[p]).wait()

        fire_uv(0, 0)

        @pl.when(nj > 1)
        def _():
            fire_uv(1, 1)

        wait_uv(0)
        phase1_and_fire(0)

        def step(j, carry):
            p = lax.rem(j, 2)
            pn = lax.rem(j + 1, 2)

            @pl.when(j + 2 < nj)
            def _():
                fire_uv(j + 2, p)

            @pl.when(j + 1 < nj)
            def _():
                wait_uv(pn)
                phase1_and_fire(pn)

            wait_gathers(p)

            @pl.when(j >= 2)
            def _():
                wait_out(p)

            phase2(p)
            fire_out(j, p)
            return carry

        lax.fori_loop(0, nj, step, 0, unroll=False)

        @pl.when(nj >= 2)
        def _():
            wait_out(lax.rem(nj, 2))

        wait_out(lax.rem(nj + 1, 2))

    return body


def kernel(uvwqList, array4D):
    n = uvwqList.shape[0]
    nchunks = -(-n // C)
    assert n >= C and n % 8 == 0

    table = array4D.reshape(NROWS * CHN // 128, 128).reshape(NROWS, CHN)
    uvwq_flat = uvwqList.reshape(-1)

    mesh = plsc.VectorSubcoreMesh(core_axis_name="c", subcore_axis_name="s")
    out = pl.kernel(
        _make_body(nchunks, n),
        mesh=mesh,
        compiler_params=pltpu.CompilerParams(
            use_tc_tiling_on_sc=False, needs_layout_passes=False),
        out_type=jax.ShapeDtypeStruct((n, CHN), jnp.float32),
        scratch_types=[
            pltpu.VMEM((2, C * 4), jnp.float32),
            pltpu.VMEM((2, 16, C), jnp.int32),
            pltpu.VMEM((2, 16, C), jnp.float32),
            pltpu.VMEM((2, 16, C, CHN), jnp.float32),
            pltpu.VMEM((2, C, CHN), jnp.float32),
            pltpu.SemaphoreType.DMA((2,)),
            pltpu.SemaphoreType.DMA((2,)),
            pltpu.SemaphoreType.DMA((2,)),
        ],
    )(uvwq_flat, table)
    return out


# speedup vs baseline: 2.8419x; 2.4399x over previous
import jax
import jax.numpy as jnp
from jax import lax
from jax.experimental import pallas as pl
from jax.experimental.pallas import tpu as pltpu
from jax.experimental.pallas import tpu_sc as plsc

DIM = 48
CHN = 8
NROWS = DIM * DIM * DIM * DIM
C = 128

NUM_CORES = 2
NUM_SUBCORES = 16
LANES = 16
NW = NUM_CORES * NUM_SUBCORES

_CORNERS = [(o3, o2, o1, o0)
            for o3 in (0, 1) for o2 in (0, 1) for o1 in (0, 1) for o0 in (0, 1)]
_OFFS = [((o3 * DIM + o2) * DIM + o1) * DIM + o0 for (o3, o2, o1, o0) in _CORNERS]


def _make_body(nchunks, n):
    def body(uvwq_hbm, table_hbm, out_hbm, uv_v, idx_v, w_v, rows_v, out_v,
             sem_uv, sem_g, sem_out):
        wid = lax.axis_index("s") * NUM_CORES + lax.axis_index("c")
        nj = (nchunks - wid + NW - 1) // NW
        iota = lax.iota(jnp.int32, LANES)

        def qbase_of(j):
            t = wid + j * NW
            return jnp.minimum(t * C, n - C)

        def fire_uv(j, p):
            pltpu.async_copy(uvwq_hbm.at[pl.ds(qbase_of(j) * 4, C * 4)],
                             uv_v.at[p], sem_uv.at[p])

        def wait_uv(p):
            pltpu.make_async_copy(uvwq_hbm.at[pl.ds(0, C * 4)],
                                  uv_v.at[p], sem_uv.at[p]).wait()

        def phase1_and_fire(p):
            def phase1(j2, carry1):
                sl = pl.ds(j2 * LANES, LANES)
                qidx = j2 * LANES + iota
                f = [
                    plsc.load_gather(
                        uv_v.at[p],
                        [qidx * 4 + jnp.full((LANES,), d, jnp.int32)])
                    * float(DIM - 1)
                    for d in range(4)
                ]
                ii = [fd.astype(jnp.int32) for fd in f]
                fr = [f[d] - ii[d].astype(jnp.float32) for d in range(4)]
                r0 = ((ii[3] * DIM + ii[2]) * DIM + ii[1]) * DIM + ii[0]
                t = [(1.0 - fr[d], fr[d]) for d in range(4)]
                for c, (o3, o2, o1, o0) in enumerate(_CORNERS):
                    idx_v[p, c, sl] = r0 + _OFFS[c]
                    w_v[p, c, sl] = (t[3][o3] * t[2][o2]) * (t[1][o1] * t[0][o0])
                return carry1

            lax.fori_loop(0, C // LANES, phase1, 0, unroll=False)
            for c in range(16):
                pltpu.async_copy(table_hbm.at[idx_v.at[p, c]],
                                 rows_v.at[p, c], sem_g.at[p])

        def wait_gathers(p):
            for c in range(16):
                pltpu.make_async_copy(table_hbm.at[idx_v.at[p, c]],
                                      rows_v.at[p, c], sem_g.at[p]).wait()

        def phase2(p):
            def inner(j2, carry2):
                sl = pl.ds(j2 * LANES, LANES)
                qidx = j2 * LANES + iota
                wv = [w_v[p, c, sl] for c in range(16)]
                for ch in range(CHN):
                    chv = jnp.full((LANES,), ch, jnp.int32)
                    acc = None
                    for c in range(16):
                        cc = jnp.full((LANES,), c, jnp.int32)
                        x = plsc.load_gather(rows_v.at[p], [cc, qidx, chv])
                        acc = wv[c] * x if acc is None else acc + wv[c] * x
                    plsc.store_scatter(out_v.at[p], [qidx, chv], acc)
                return carry2

            lax.fori_loop(0, C // LANES, inner, 0, unroll=False)

        def fire_out(j, p):
            pltpu.async_copy(out_v.at[p], out_hbm.at[pl.ds(qbase_of(j), C)],
                             sem_out.at[p])

        def wait_out(p):
            pltpu.make_async_copy(out_v.at[p], out_hbm.at[pl.ds(0, C)],
                                  sem_out.at[p]).wait()

        fire_uv(0, 0)

        @pl.when(nj > 1)
        def _():
            fire_uv(1, 1)

        wait_uv(0)
        phase1_and_fire(0)

        def step(j, carry):
            p = lax.rem(j, 2)
            pn = lax.rem(j + 1, 2)

            @pl.when(j + 2 < nj)
            def _():
                fire_uv(j + 2, p)

            @pl.when(j + 1 < nj)
            def _():
                wait_uv(pn)
                phase1_and_fire(pn)

            wait_gathers(p)

            @pl.when(j >= 2)
            def _():
                wait_out(p)

            phase2(p)
            fire_out(j, p)
            return carry

        lax.fori_loop(0, nj, step, 0, unroll=False)

        @pl.when(nj >= 2)
        def _():
            wait_out(lax.rem(nj, 2))

        wait_out(lax.rem(nj + 1, 2))

    return body


def kernel(uvwqList, array4D):
    n = uvwqList.shape[0]
    nchunks = -(-n // C)
    assert n >= C and n % 8 == 0

    eye384, eye128 = lax.optimization_barrier(
        (jnp.eye(384, dtype=jnp.float32), jnp.eye(128, dtype=jnp.float32)))
    w_t = eye384.reshape(384, DIM, CHN)
    a3 = array4D.reshape(DIM * DIM * DIM, DIM, CHN)
    table = jnp.einsum(
        "cio,xio->cx", a3, w_t, precision=lax.Precision.HIGHEST,
    ).reshape(NROWS, CHN)

    w_q = eye128.reshape(128, 32, 4)
    q3 = uvwqList.reshape(n // 32, 32, 4)
    uvwq_flat = jnp.einsum(
        "rld,xld->rx", q3, w_q, precision=lax.Precision.HIGHEST,
    ).reshape(-1)

    mesh = plsc.VectorSubcoreMesh(core_axis_name="c", subcore_axis_name="s")
    out = pl.kernel(
        _make_body(nchunks, n),
        mesh=mesh,
        compiler_params=pltpu.CompilerParams(
            use_tc_tiling_on_sc=False, needs_layout_passes=False),
        out_type=jax.ShapeDtypeStruct((n, CHN), jnp.float32),
        scratch_types=[
            pltpu.VMEM((2, C * 4), jnp.float32),
            pltpu.VMEM((2, 16, C), jnp.int32),
            pltpu.VMEM((2, 16, C), jnp.float32),
            pltpu.VMEM((2, 16, C, CHN), jnp.float32),
            pltpu.VMEM((2, C, CHN), jnp.float32),
            pltpu.SemaphoreType.DMA((2,)),
            pltpu.SemaphoreType.DMA((2,)),
            pltpu.SemaphoreType.DMA((2,)),
        ],
    )(uvwq_flat, table)
    return out


# speedup vs baseline: 3.1180x; 1.0972x over previous
import jax
import jax.numpy as jnp
from jax import lax
from jax.experimental import pallas as pl
from jax.experimental.pallas import tpu as pltpu
from jax.experimental.pallas import tpu_sc as plsc

DIM = 48
CHN = 8
NROWS = DIM * DIM * DIM * DIM
C = 128

NUM_CORES = 2
NUM_SUBCORES = 16
LANES = 16
NW = NUM_CORES * NUM_SUBCORES

_CORNERS = [(o3, o2, o1, o0)
            for o3 in (0, 1) for o2 in (0, 1) for o1 in (0, 1) for o0 in (0, 1)]
_OFFS = [((o3 * DIM + o2) * DIM + o1) * DIM + o0 for (o3, o2, o1, o0) in _CORNERS]


def _make_body(nchunks, n):
    def body(uvwq_hbm, table_hbm, out_hbm, uv_v, idx_v, w_v, rows_v, out_v,
             sem_uv, sem_g, sem_out):
        wid = lax.axis_index("s") * NUM_CORES + lax.axis_index("c")
        nj = (nchunks - wid + NW - 1) // NW
        iota = lax.iota(jnp.int32, LANES)

        def qbase_of(j):
            t = wid + j * NW
            return jnp.minimum(t * C, n - C)

        def fire_uv(j, p):
            pltpu.async_copy(uvwq_hbm.at[pl.ds(qbase_of(j) * 4, C * 4)],
                             uv_v.at[p], sem_uv.at[p])

        def wait_uv(p):
            pltpu.make_async_copy(uvwq_hbm.at[pl.ds(0, C * 4)],
                                  uv_v.at[p], sem_uv.at[p]).wait()

        def phase1_and_fire(p):
            def phase1(j2, carry1):
                sl = pl.ds(j2 * LANES, LANES)
                qidx = j2 * LANES + iota
                f = [
                    plsc.load_gather(
                        uv_v.at[p],
                        [qidx * 4 + jnp.full((LANES,), d, jnp.int32)])
                    * float(DIM - 1)
                    for d in range(4)
                ]
                ii = [fd.astype(jnp.int32) for fd in f]
                fr = [f[d] - ii[d].astype(jnp.float32) for d in range(4)]
                r0 = ((ii[3] * DIM + ii[2]) * DIM + ii[1]) * DIM + ii[0]
                t = [(1.0 - fr[d], fr[d]) for d in range(4)]
                for c, (o3, o2, o1, o0) in enumerate(_CORNERS):
                    idx_v[p, c, sl] = r0 + _OFFS[c]
                    w_v[p, c, sl] = (t[3][o3] * t[2][o2]) * (t[1][o1] * t[0][o0])
                return carry1

            lax.fori_loop(0, C // LANES, phase1, 0, unroll=False)
            for c in range(16):
                pltpu.async_copy(table_hbm.at[idx_v.at[p, c]],
                                 rows_v.at[p, c], sem_g.at[p])

        def wait_gathers(p):
            for c in range(16):
                pltpu.make_async_copy(table_hbm.at[idx_v.at[p, c]],
                                      rows_v.at[p, c], sem_g.at[p]).wait()

        def phase2(p):
            def inner(j2, carry2):
                sl = pl.ds(j2 * LANES, LANES)
                qidx = j2 * LANES + iota
                wv = [w_v[p, c, sl] for c in range(16)]
                for ch in range(CHN):
                    chv = jnp.full((LANES,), ch, jnp.int32)
                    acc = None
                    for c in range(16):
                        cc = jnp.full((LANES,), c, jnp.int32)
                        x = plsc.load_gather(rows_v.at[p], [cc, qidx, chv])
                        acc = wv[c] * x if acc is None else acc + wv[c] * x
                    plsc.store_scatter(out_v.at[p], [qidx, chv], acc)
                return carry2

            lax.fori_loop(0, C // LANES, inner, 0, unroll=False)

        def fire_out(j, p):
            pltpu.async_copy(out_v.at[p], out_hbm.at[pl.ds(qbase_of(j), C)],
                             sem_out.at[p])

        def wait_out(p):
            pltpu.make_async_copy(out_v.at[p], out_hbm.at[pl.ds(0, C)],
                                  sem_out.at[p]).wait()

        fire_uv(0, 0)

        @pl.when(nj > 1)
        def _():
            fire_uv(1, 1)

        wait_uv(0)
        phase1_and_fire(0)

        def step(j, carry):
            p = lax.rem(j, 2)
            pn = lax.rem(j + 1, 2)

            @pl.when(j + 2 < nj)
            def _():
                fire_uv(j + 2, p)

            @pl.when(j + 1 < nj)
            def _():
                wait_uv(pn)
                phase1_and_fire(pn)

            wait_gathers(p)

            @pl.when(j >= 2)
            def _():
                wait_out(p)

            phase2(p)
            fire_out(j, p)
            return carry

        lax.fori_loop(0, nj, step, 0, unroll=False)

        @pl.when(nj >= 2)
        def _():
            wait_out(lax.rem(nj, 2))

        wait_out(lax.rem(nj + 1, 2))

    return body


def kernel(uvwqList, array4D):
    n = uvwqList.shape[0]
    nchunks = -(-n // C)
    assert n >= C and n % 8 == 0

    eye384, eye128 = lax.optimization_barrier(
        (jnp.eye(384, dtype=jnp.float32), jnp.eye(128, dtype=jnp.float32)))
    w_t = eye384.reshape(384, DIM, CHN).transpose(0, 2, 1)
    a3 = jnp.transpose(array4D, (0, 1, 2, 4, 3)).reshape(
        DIM * DIM * DIM, CHN, DIM)
    table = jnp.einsum(
        "coi,xoi->cx", a3, w_t, precision=lax.Precision.HIGH,
    ).reshape(NROWS, CHN)

    w_q = eye128.reshape(128, 32, 4)
    q3 = uvwqList.reshape(n // 32, 32, 4)
    uvwq_flat = jnp.einsum(
        "rld,xld->rx", q3, w_q, precision=lax.Precision.HIGH,
    ).reshape(-1)

    mesh = plsc.VectorSubcoreMesh(core_axis_name="c", subcore_axis_name="s")
    out = pl.kernel(
        _make_body(nchunks, n),
        mesh=mesh,
        compiler_params=pltpu.CompilerParams(
            use_tc_tiling_on_sc=False, needs_layout_passes=False),
        out_type=jax.ShapeDtypeStruct((n, CHN), jnp.float32),
        scratch_types=[
            pltpu.VMEM((2, C * 4), jnp.float32),
            pltpu.VMEM((2, 16, C), jnp.int32),
            pltpu.VMEM((2, 16, C), jnp.float32),
            pltpu.VMEM((2, 16, C, CHN), jnp.float32),
            pltpu.VMEM((2, C, CHN), jnp.float32),
            pltpu.SemaphoreType.DMA((2,)),
            pltpu.SemaphoreType.DMA((2,)),
            pltpu.SemaphoreType.DMA((2,)),
        ],
    )(uvwq_flat, table)
    return out
